# baseline (speedup 1.0000x reference)
import functools

import jax
import jax.numpy as jnp
from jax import lax
from jax.experimental import pallas as pl
from jax.experimental.pallas import tpu as pltpu
from jax.experimental.pallas import tpu_sc as plsc

VOCAB = 1000000
DIM = 64
HALF = VOCAB // 2

NC = 2
NS = 16
NW = NC * NS

TBLK = 512
NFULL = VOCAB // TBLK
TAIL = VOCAB - NFULL * TBLK
BLOCKS_PER_W = (NFULL + NW - 1) // NW

_params = pltpu.CompilerParams(
    use_tc_tiling_on_sc=True, needs_layout_passes=False
)
_mesh = lambda: plsc.VectorSubcoreMesh(core_axis_name="c", subcore_axis_name="s")


@functools.partial(
    pl.kernel,
    out_type=jax.ShapeDtypeStruct((HALF, 2 * DIM), jnp.float32),
    mesh=_mesh(),
    scratch_types=[
        pltpu.VMEM((DIM, TBLK), jnp.float32),
        pltpu.VMEM((TBLK // 2, 2 * DIM), jnp.float32),
        pltpu.VMEM((DIM, TAIL), jnp.float32),
        pltpu.VMEM((TAIL // 2, 2 * DIM), jnp.float32),
    ],
    compiler_params=_params,
)
def _transpose_kernel(wt, pairs, tin, tout, tin2, tout2):
    wid = lax.axis_index("s") * NC + lax.axis_index("c")

    def transpose_block(src, dst, n):
        @pl.loop(0, n // 2, step=16)
        def rows(p0):
            for pp in range(16):
                p = p0 + pp
                for k in range(2 * DIM // 16):
                    q = jax.lax.broadcasted_iota(jnp.int32, (16,), 0) + 16 * k
                    dvec = q % DIM
                    cvec = 2 * p + q // DIM
                    dst[p, pl.ds(16 * k, 16)] = plsc.load_gather(src, [dvec, cvec])

    @pl.loop(0, BLOCKS_PER_W)
    def per_block(j):
        i = wid + NW * j

        @pl.when(i < NFULL)
        def _():
            c0 = pl.multiple_of(i * TBLK, TBLK)
            pltpu.sync_copy(wt.at[:, pl.ds(c0, TBLK)], tin)
            transpose_block(tin, tout, TBLK)
            pltpu.sync_copy(
                tout, pairs.at[pl.ds(pl.multiple_of(i * (TBLK // 2), TBLK // 2), TBLK // 2)]
            )

    @pl.when(wid == 1)
    def _():
        c0 = NFULL * TBLK
        pltpu.sync_copy(wt.at[:, pl.ds(c0, TAIL)], tin2)
        transpose_block(tin2, tout2, TAIL)
        pltpu.sync_copy(tout2, pairs.at[pl.ds(c0 // 2, TAIL // 2)])


def _make_gather(bsz: int, l: int):
    bb = bsz // NW

    @functools.partial(
        pl.kernel,
        out_type=jax.ShapeDtypeStruct((l, DIM, bsz), jnp.float32),
        mesh=_mesh(),
        scratch_types=[
            pltpu.VMEM((l, bb), jnp.int32),
            pltpu.VMEM((bb,), jnp.int32),
            pltpu.VMEM((bb, 2 * DIM), jnp.float32),
            pltpu.VMEM((DIM, bb), jnp.float32),
            pltpu.SemaphoreType.DMA,
        ],
        compiler_params=_params,
    )
    def gather(pairs, idx_t, out, idx_v, gidx, rows, slab, sem):
        wid = lax.axis_index("s") * NC + lax.axis_index("c")
        b0 = pl.multiple_of(wid * bb, bb)
        pltpu.sync_copy(idx_t.at[:, pl.ds(b0, bb)], idx_v)

        @pl.loop(0, l)
        def per_l(li):
            for j in range(bb // 16):
                r = idx_v[li, pl.ds(16 * j, 16)]
                gidx[pl.ds(16 * j, 16)] = r >> 1
            pltpu.async_copy(pairs.at[gidx], rows, sem).wait()
            for j in range(bb // 16):
                r = idx_v[li, pl.ds(16 * j, 16)]
                base = jax.lax.broadcasted_iota(jnp.int32, (16,), 0) + 16 * j
                coff = (r & 1) * DIM
                for d in range(DIM):
                    slab[d, pl.ds(16 * j, 16)] = plsc.load_gather(
                        rows, [base, coff + d]
                    )
            pltpu.sync_copy(slab, out.at[li, :, pl.ds(b0, bb)])

    return gather


def kernel(input, weight):
    bsz, l = input.shape
    pairs = _transpose_kernel(weight.T)
    out = _make_gather(bsz, l)(pairs, input.T.astype(jnp.int32))
    return out.transpose(2, 0, 1)


# speedup vs baseline: 1.5634x; 1.5634x over previous
import functools

import jax
import jax.numpy as jnp
from jax import lax
from jax.experimental import pallas as pl
from jax.experimental.pallas import tpu as pltpu
from jax.experimental.pallas import tpu_sc as plsc

VOCAB = 1000000
DIM = 64

NC = 2
NS = 16
NW = NC * NS

_params = pltpu.CompilerParams(
    use_tc_tiling_on_sc=True, needs_layout_passes=False
)


def _make_gather(bsz: int, l: int):
    bb = bsz // NW
    mesh = plsc.VectorSubcoreMesh(core_axis_name="c", subcore_axis_name="s")

    @functools.partial(
        pl.kernel,
        out_type=jax.ShapeDtypeStruct((l, DIM, bsz), jnp.float32),
        mesh=mesh,
        scratch_types=[
            pltpu.VMEM((l, bb), jnp.int32),
            pltpu.VMEM((bb,), jnp.int32),
            pltpu.VMEM((bb, 2 * DIM), jnp.float32),
            pltpu.VMEM((DIM, bb), jnp.float32),
            pltpu.SemaphoreType.DMA,
        ],
        compiler_params=_params,
    )
    def gather(table, idx_t, out, idx_v, gidx, rows, slab, sem):
        wid = lax.axis_index("s") * NC + lax.axis_index("c")
        b0 = pl.multiple_of(wid * bb, bb)
        pltpu.sync_copy(idx_t.at[:, pl.ds(b0, bb)], idx_v)

        @pl.loop(0, l)
        def per_l(li):
            for j in range(bb // 16):
                gidx[pl.ds(16 * j, 16)] = idx_v[li, pl.ds(16 * j, 16)]
            pltpu.async_copy(table.at[gidx], rows, sem).wait()
            bases = [
                jax.lax.broadcasted_iota(jnp.int32, (16,), 0) + 16 * j
                for j in range(bb // 16)
            ]
            for d in range(DIM):
                dvec = jnp.full((16,), d, jnp.int32)
                for j in range(bb // 16):
                    slab[d, pl.ds(16 * j, 16)] = plsc.load_gather(
                        rows, [bases[j], dvec]
                    )
            pltpu.sync_copy(slab, out.at[li, :, pl.ds(b0, bb)])

    return gather


def kernel(input, weight):
    bsz, l = input.shape
    wpad = jnp.pad(weight, ((0, 0), (0, 128 - DIM)))
    out = _make_gather(bsz, l)(wpad, input.T.astype(jnp.int32))
    return out.transpose(2, 0, 1)


# speedup vs baseline: 2.4740x; 1.5824x over previous
import functools

import jax
import jax.numpy as jnp
from jax import lax
from jax.experimental import pallas as pl
from jax.experimental.pallas import tpu as pltpu
from jax.experimental.pallas import tpu_sc as plsc

VOCAB = 1000000
DIM = 64

NC = 2
NS = 16
NW = NC * NS

_params = pltpu.CompilerParams(
    use_tc_tiling_on_sc=True, needs_layout_passes=False
)


def _make_gather(bsz: int, l: int):
    bb = bsz // NW
    mesh = plsc.VectorSubcoreMesh(core_axis_name="c", subcore_axis_name="s")

    @functools.partial(
        pl.kernel,
        out_type=jax.ShapeDtypeStruct((l, DIM, bsz), jnp.float32),
        mesh=mesh,
        scratch_types=[
            pltpu.VMEM((l, bb), jnp.int32),
            [pltpu.VMEM((bb,), jnp.int32) for _ in range(2)],
            [pltpu.VMEM((bb, 2 * DIM), jnp.float32) for _ in range(2)],
            pltpu.VMEM((DIM, bb), jnp.float32),
            [pltpu.SemaphoreType.DMA for _ in range(2)],
        ],
        compiler_params=_params,
    )
    def gather(table, idx_t, out, idx_v, gidx, rows, slab, sem):
        wid = lax.axis_index("s") * NC + lax.axis_index("c")
        b0 = pl.multiple_of(wid * bb, bb)
        pltpu.sync_copy(idx_t.at[:, pl.ds(b0, bb)], idx_v)

        bases = [
            jax.lax.broadcasted_iota(jnp.int32, (16,), 0) + 16 * j
            for j in range(bb // 16)
        ]

        def fire(li, k):
            for j in range(bb // 16):
                gidx[k][pl.ds(16 * j, 16)] = idx_v[li, pl.ds(16 * j, 16)]
            pltpu.async_copy(table.at[gidx[k]], rows[k], sem[k])

        def wait(k):
            pltpu.make_async_copy(table.at[pl.ds(0, bb)], rows[k], sem[k]).wait()

        def drain(li, k):
            for j in range(bb // 16):
                for d0 in range(0, DIM, 8):
                    vals = [
                        plsc.load_gather(
                            rows[k],
                            [bases[j], jnp.full((16,), d0 + t, jnp.int32)],
                        )
                        for t in range(8)
                    ]
                    for t in range(8):
                        slab[d0 + t, pl.ds(16 * j, 16)] = vals[t]
            pltpu.sync_copy(slab, out.at[li, :, pl.ds(b0, bb)])

        fire(0, 0)

        @pl.loop(0, l // 2)
        def per_pair(h):
            l0 = 2 * h
            fire(l0 + 1, 1)
            wait(0)
            drain(l0, 0)

            @pl.when(l0 + 2 < l)
            def _():
                fire(l0 + 2, 0)

            wait(1)
            drain(l0 + 1, 1)

    return gather


def kernel(input, weight):
    bsz, l = input.shape
    wpad = jnp.pad(weight, ((0, 0), (0, 128 - DIM)))
    out = _make_gather(bsz, l)(wpad, input.T.astype(jnp.int32))
    return out.transpose(2, 0, 1)


# speedup vs baseline: 2.4921x; 1.0073x over previous
import functools

import jax
import jax.numpy as jnp
from jax import lax
from jax.experimental import pallas as pl
from jax.experimental.pallas import tpu as pltpu
from jax.experimental.pallas import tpu_sc as plsc

VOCAB = 1000000
DIM = 64

NC = 2
NS = 16
NW = NC * NS

_params = pltpu.CompilerParams(
    use_tc_tiling_on_sc=True, needs_layout_passes=False
)


def _make_gather(bsz: int, l: int):
    bb = bsz // NW
    mesh = plsc.VectorSubcoreMesh(core_axis_name="c", subcore_axis_name="s")

    @functools.partial(
        pl.kernel,
        out_type=jax.ShapeDtypeStruct((l, DIM, bsz), jnp.float32),
        mesh=mesh,
        scratch_types=[
            pltpu.VMEM((l, bb), jnp.int32),
            [pltpu.VMEM((bb,), jnp.int32) for _ in range(2)],
            [pltpu.VMEM((bb, 2 * DIM), jnp.float32) for _ in range(2)],
            pltpu.VMEM((DIM, bb), jnp.float32),
            [pltpu.SemaphoreType.DMA for _ in range(2)],
        ],
        compiler_params=_params,
    )
    def gather(table, idx_t, out, idx_v, gidx, rows, slab, sem):
        wid = lax.axis_index("s") * NC + lax.axis_index("c")
        b0 = pl.multiple_of(wid * bb, bb)
        pltpu.sync_copy(idx_t.at[:, pl.ds(b0, bb)], idx_v)

        bases = [
            jax.lax.broadcasted_iota(jnp.int32, (16,), 0) + 16 * j
            for j in range(bb // 16)
        ]

        def fire(li, k):
            for j in range(bb // 16):
                gidx[k][pl.ds(16 * j, 16)] = idx_v[li, pl.ds(16 * j, 16)]
            pltpu.async_copy(table.at[gidx[k]], rows[k], sem[k])

        def wait(k):
            pltpu.make_async_copy(table.at[pl.ds(0, bb)], rows[k], sem[k]).wait()

        def drain(li, k):
            for j in range(bb // 16):
                for d0 in range(0, DIM, 16):
                    vals = [
                        plsc.load_gather(
                            rows[k],
                            [bases[j], jnp.full((16,), d0 + t, jnp.int32)],
                        )
                        for t in range(16)
                    ]
                    for t in range(16):
                        slab[d0 + t, pl.ds(16 * j, 16)] = vals[t]
            pltpu.sync_copy(slab, out.at[li, :, pl.ds(b0, bb)])

        fire(0, 0)

        @pl.loop(0, l // 2)
        def per_pair(h):
            l0 = 2 * h
            fire(l0 + 1, 1)
            wait(0)
            drain(l0, 0)

            @pl.when(l0 + 2 < l)
            def _():
                fire(l0 + 2, 0)

            wait(1)
            drain(l0 + 1, 1)

    return gather


def kernel(input, weight):
    bsz, l = input.shape
    wpad = jnp.pad(weight, ((0, 0), (0, 128 - DIM)))
    out = _make_gather(bsz, l)(wpad, input.T.astype(jnp.int32))
    return out.transpose(2, 0, 1)


# speedup vs baseline: 2.7009x; 1.0838x over previous
import functools

import jax
import jax.numpy as jnp
from jax import lax
from jax.experimental import pallas as pl
from jax.experimental.pallas import tpu as pltpu
from jax.experimental.pallas import tpu_sc as plsc

VOCAB = 1000000
DIM = 64

NC = 2
NS = 16
NW = NC * NS

CHUNK = 128
NBUF = 4


def _make_lookup(n_idx: int):
    n_rows = n_idx // CHUNK
    rows_per_w = n_rows // NW
    assert rows_per_w % NBUF == 0 and rows_per_w >= 2 * NBUF

    mesh = plsc.VectorSubcoreMesh(core_axis_name="c", subcore_axis_name="s")

    @functools.partial(
        pl.kernel,
        out_type=jax.ShapeDtypeStruct((n_idx, DIM), jnp.float32),
        mesh=mesh,
        scratch_types=[
            pltpu.VMEM((rows_per_w, CHUNK), jnp.int32),
            [pltpu.VMEM((CHUNK, DIM), jnp.float32) for _ in range(NBUF)],
            [pltpu.SemaphoreType.DMA for _ in range(NBUF)],
            [pltpu.SemaphoreType.DMA for _ in range(NBUF)],
        ],
        compiler_params=pltpu.CompilerParams(use_tc_tiling_on_sc=False),
    )
    def lookup(table_hbm, idx_hbm, out_hbm, idx_v, rows, gsem, wsem):
        wid = lax.axis_index("s") * NC + lax.axis_index("c")
        base_row = wid * rows_per_w
        pltpu.sync_copy(idx_hbm.at[pl.ds(base_row, rows_per_w)], idx_v)

        def gather_start(g, b):
            pltpu.async_copy(table_hbm.at[idx_v.at[g]], rows[b], gsem[b])

        def gather_wait(b):
            pltpu.make_async_copy(
                table_hbm.at[pl.ds(0, CHUNK)], rows[b], gsem[b]
            ).wait()

        def out_slice(g):
            return out_hbm.at[pl.ds((base_row + g) * CHUNK, CHUNK)]

        def writeback_start(g, b):
            return pltpu.async_copy(rows[b], out_slice(g), wsem[b])

        for b in range(NBUF):
            gather_start(b, b)

        @pl.loop(0, rows_per_w - NBUF, step=NBUF)
        def body(g0):
            for b in range(NBUF):
                g = g0 + b
                gather_wait(b)
                writeback_start(g, b).wait()
                gather_start(g + NBUF, b)

        for b in range(NBUF):
            g = rows_per_w - NBUF + b
            gather_wait(b)
            writeback_start(g, b)
        for b in range(NBUF):
            g = rows_per_w - NBUF + b
            pltpu.make_async_copy(rows[b], out_slice(g), wsem[b]).wait()

    return lookup


def kernel(input, weight):
    b, l = input.shape
    n_idx = b * l
    idx2d = input.reshape(n_idx // CHUNK, CHUNK).astype(jnp.int32)
    out = _make_lookup(n_idx)(weight, idx2d)
    return out.reshape(b, l, DIM)
